# baseline (speedup 1.0000x reference)
import numpy as np
import jax
import jax.numpy as jnp
from jax.experimental import pallas as pl
from jax.experimental.pallas import tpu as pltpu

_EPS = 1e-5


def _bilinear_matrix(n_in: int, n_out: int) -> np.ndarray:
    A = np.zeros((n_out, n_in), dtype=np.float32)
    if n_in == 1 or n_out == 1:
        A[:, 0] = 1.0
        return A
    src = np.arange(n_out, dtype=np.float64) * (n_in - 1) / (n_out - 1)
    lo = np.clip(np.floor(src).astype(np.int64), 0, n_in - 2)
    frac = (src - lo).astype(np.float32)
    A[np.arange(n_out), lo] += 1.0 - frac
    A[np.arange(n_out), lo + 1] += frac
    return A


def _mix_moments_kernel(x_ref, w_ref, z_ref, s_ref, q_ref):
    X = x_ref[0]
    Z = jnp.dot(w_ref[...], X, preferred_element_type=jnp.float32)
    z_ref[0] = Z
    s_ref[0] = jnp.sum(Z, axis=1, keepdims=True)
    q_ref[0] = jnp.sum(Z * Z, axis=1, keepdims=True)


def _upsample_affine_kernel(z_ref, awt_ref, ah_ref, sc_ref, sh_ref, o_ref):
    Ho, H = ah_ref.shape
    Wo = awt_ref.shape[1]
    C = sc_ref.shape[0]

    zr = z_ref[0]
    t = jnp.dot(zr, awt_ref[...], preferred_element_type=jnp.float32)
    t3 = t.reshape(C, H, Wo)
    ah = jnp.broadcast_to(ah_ref[...][None, :, :], (C, Ho, H))
    u = jnp.einsum('cyh,chv->cyv', ah, t3,
                   preferred_element_type=jnp.float32)
    u2 = u.reshape(C, Ho * Wo)
    o_ref[0] = (u2 * sc_ref[...] + sh_ref[...]).astype(o_ref.dtype)


def kernel(x_nchw, conv_w, conv_b, bn_gamma, bn_beta):
    del conv_b
    N, C_in, H, W = x_nchw.shape
    C_out = conv_w.shape[0]
    factor = 2
    Ho, Wo = H * factor, W * factor
    cnt = N * H * W

    x = x_nchw.astype(jnp.float32).reshape(N, C_in, H * W)
    W2 = conv_w.reshape(C_out, C_in).astype(jnp.float32)

    Z, S, Q = pl.pallas_call(
        _mix_moments_kernel,
        out_shape=(jax.ShapeDtypeStruct((N, C_out, H * W), jnp.float32),
                   jax.ShapeDtypeStruct((N, C_out, 1), jnp.float32),
                   jax.ShapeDtypeStruct((N, C_out, 1), jnp.float32)),
        grid=(N,),
        in_specs=[pl.BlockSpec((1, C_in, H * W), lambda n: (n, 0, 0)),
                  pl.BlockSpec((C_out, C_in), lambda n: (0, 0))],
        out_specs=(pl.BlockSpec((1, C_out, H * W), lambda n: (n, 0, 0)),
                   pl.BlockSpec((1, C_out, 1), lambda n: (n, 0, 0)),
                   pl.BlockSpec((1, C_out, 1), lambda n: (n, 0, 0))),
        compiler_params=pltpu.CompilerParams(dimension_semantics=("parallel",)),
    )(x, W2)

    mean = jnp.sum(S, axis=0)[:, 0] / cnt
    var = jnp.maximum(jnp.sum(Q, axis=0)[:, 0] / cnt - mean * mean, 0.0)
    scale = bn_gamma.astype(jnp.float32) * jax.lax.rsqrt(var + _EPS)
    shift = bn_beta.astype(jnp.float32) - scale * mean

    A_h = jnp.asarray(_bilinear_matrix(H, Ho))
    A_wT = jnp.asarray(_bilinear_matrix(W, Wo).T)

    Zr = Z.reshape(N, C_out * H, W)
    flops = 2 * N * (C_out * H * W * Wo + C_out * Ho * H * Wo)
    bytes_accessed = 4 * (Z.size + N * C_out * Ho * Wo)

    out_flat = pl.pallas_call(
        _upsample_affine_kernel,
        out_shape=jax.ShapeDtypeStruct((N, C_out, Ho * Wo), jnp.float32),
        grid=(N,),
        in_specs=[
            pl.BlockSpec((1, C_out * H, W), lambda n: (n, 0, 0)),
            pl.BlockSpec((W, Wo), lambda n: (0, 0)),
            pl.BlockSpec((Ho, H), lambda n: (0, 0)),
            pl.BlockSpec((C_out, 1), lambda n: (0, 0)),
            pl.BlockSpec((C_out, 1), lambda n: (0, 0)),
        ],
        out_specs=pl.BlockSpec((1, C_out, Ho * Wo), lambda n: (n, 0, 0)),
        compiler_params=pltpu.CompilerParams(
            dimension_semantics=("parallel",),
            vmem_limit_bytes=48 * 1024 * 1024),
        cost_estimate=pl.CostEstimate(flops=flops, transcendentals=0,
                                      bytes_accessed=bytes_accessed),
    )(Zr, A_wT, A_h, scale[:, None], shift[:, None])

    return out_flat.reshape(N, C_out, Ho, Wo)


# speedup vs baseline: 1.1369x; 1.1369x over previous
import numpy as np
import jax
import jax.numpy as jnp
from jax.experimental import pallas as pl
from jax.experimental.pallas import tpu as pltpu

_EPS = 1e-5


def _bilinear_matrix(n_in: int, n_out: int) -> np.ndarray:
    A = np.zeros((n_out, n_in), dtype=np.float32)
    if n_in == 1 or n_out == 1:
        A[:, 0] = 1.0
        return A
    src = np.arange(n_out, dtype=np.float64) * (n_in - 1) / (n_out - 1)
    lo = np.clip(np.floor(src).astype(np.int64), 0, n_in - 2)
    frac = (src - lo).astype(np.float32)
    A[np.arange(n_out), lo] += 1.0 - frac
    A[np.arange(n_out), lo + 1] += frac
    return A


def _mix_moments_kernel(x_ref, w_ref, z_ref, s_ref, q_ref):
    nb = x_ref.shape[0]
    for i in range(nb):
        Xb = x_ref[i].astype(jnp.bfloat16)
        Z = jnp.dot(w_ref[...], Xb, preferred_element_type=jnp.float32)
        z_ref[i] = Z.astype(jnp.bfloat16)
        s_ref[i] = jnp.sum(Z, axis=1, keepdims=True)
        q_ref[i] = jnp.sum(Z * Z, axis=1, keepdims=True)


def _upsample_affine_kernel(z_ref, awt_ref, ah_ref, sc_ref, sh_ref, o_ref):
    Ho, H = ah_ref.shape
    Wo = awt_ref.shape[1]
    C = sc_ref.shape[0]

    zr = z_ref[0]
    t = jnp.dot(zr, awt_ref[...], preferred_element_type=jnp.float32)
    t3 = t.astype(jnp.bfloat16).reshape(C, H, Wo)
    ah = jnp.broadcast_to(ah_ref[...][None, :, :], (C, Ho, H))
    u = jnp.einsum('cyh,chv->cyv', ah, t3,
                   preferred_element_type=jnp.float32)
    u2 = u.reshape(C, Ho * Wo)
    o_ref[0] = u2 * sc_ref[...] + sh_ref[...]


def kernel(x_nchw, conv_w, conv_b, bn_gamma, bn_beta):
    del conv_b
    N, C_in, H, W = x_nchw.shape
    C_out = conv_w.shape[0]
    factor = 2
    Ho, Wo = H * factor, W * factor
    cnt = N * H * W
    NB = 8

    x = x_nchw.astype(jnp.float32).reshape(N, C_in, H * W)
    W2 = conv_w.reshape(C_out, C_in).astype(jnp.bfloat16)

    Z, S, Q = pl.pallas_call(
        _mix_moments_kernel,
        out_shape=(jax.ShapeDtypeStruct((N, C_out, H * W), jnp.bfloat16),
                   jax.ShapeDtypeStruct((N, C_out, 1), jnp.float32),
                   jax.ShapeDtypeStruct((N, C_out, 1), jnp.float32)),
        grid=(N // NB,),
        in_specs=[pl.BlockSpec((NB, C_in, H * W), lambda n: (n, 0, 0)),
                  pl.BlockSpec((C_out, C_in), lambda n: (0, 0))],
        out_specs=(pl.BlockSpec((NB, C_out, H * W), lambda n: (n, 0, 0)),
                   pl.BlockSpec((NB, C_out, 1), lambda n: (n, 0, 0)),
                   pl.BlockSpec((NB, C_out, 1), lambda n: (n, 0, 0))),
        compiler_params=pltpu.CompilerParams(
            dimension_semantics=("arbitrary",)),
    )(x, W2)

    mean = jnp.sum(S, axis=0)[:, 0] / cnt
    var = jnp.maximum(jnp.sum(Q, axis=0)[:, 0] / cnt - mean * mean, 0.0)
    scale = bn_gamma.astype(jnp.float32) * jax.lax.rsqrt(var + _EPS)
    shift = bn_beta.astype(jnp.float32) - scale * mean

    A_h = jnp.asarray(_bilinear_matrix(H, Ho)).astype(jnp.bfloat16)
    A_wT = jnp.asarray(_bilinear_matrix(W, Wo).T).astype(jnp.bfloat16)

    Zr = Z.reshape(N, C_out * H, W)
    flops = 2 * N * (C_out * H * W * Wo + C_out * Ho * H * Wo)
    bytes_accessed = 2 * Z.size + 4 * N * C_out * Ho * Wo

    out_flat = pl.pallas_call(
        _upsample_affine_kernel,
        out_shape=jax.ShapeDtypeStruct((N, C_out, Ho * Wo), jnp.float32),
        grid=(N,),
        in_specs=[
            pl.BlockSpec((1, C_out * H, W), lambda n: (n, 0, 0)),
            pl.BlockSpec((W, Wo), lambda n: (0, 0)),
            pl.BlockSpec((Ho, H), lambda n: (0, 0)),
            pl.BlockSpec((C_out, 1), lambda n: (0, 0)),
            pl.BlockSpec((C_out, 1), lambda n: (0, 0)),
        ],
        out_specs=pl.BlockSpec((1, C_out, Ho * Wo), lambda n: (n, 0, 0)),
        compiler_params=pltpu.CompilerParams(
            dimension_semantics=("arbitrary",),
            vmem_limit_bytes=48 * 1024 * 1024),
        cost_estimate=pl.CostEstimate(flops=flops, transcendentals=0,
                                      bytes_accessed=bytes_accessed),
    )(Zr, A_wT, A_h, scale[:, None], shift[:, None])

    return out_flat.reshape(N, C_out, Ho, Wo)


# speedup vs baseline: 1.1460x; 1.0080x over previous
import numpy as np
import jax
import jax.numpy as jnp
from jax.experimental import pallas as pl
from jax.experimental.pallas import tpu as pltpu

_EPS = 1e-5


def _bilinear_matrix(n_in: int, n_out: int) -> np.ndarray:
    A = np.zeros((n_out, n_in), dtype=np.float32)
    if n_in == 1 or n_out == 1:
        A[:, 0] = 1.0
        return A
    src = np.arange(n_out, dtype=np.float64) * (n_in - 1) / (n_out - 1)
    lo = np.clip(np.floor(src).astype(np.int64), 0, n_in - 2)
    frac = (src - lo).astype(np.float32)
    A[np.arange(n_out), lo] += 1.0 - frac
    A[np.arange(n_out), lo + 1] += frac
    return A


def _mix_moments_kernel(x_ref, w_ref, z_ref, s_ref, q_ref):
    nb = x_ref.shape[0]
    for i in range(nb):
        Xb = x_ref[i].astype(jnp.bfloat16)
        Z = jnp.dot(w_ref[...], Xb, preferred_element_type=jnp.float32)
        z_ref[i] = Z.astype(jnp.bfloat16)
        s_ref[i] = jnp.sum(Z, axis=1, keepdims=True)
        q_ref[i] = jnp.sum(Z * Z, axis=1, keepdims=True)


def _upsample_affine_kernel(z_ref, awt_ref, ah_ref, sc_ref, sh_ref, o_ref):
    Ho, H = ah_ref.shape
    Wo = awt_ref.shape[1]
    C = sc_ref.shape[0]

    ah = jnp.broadcast_to(ah_ref[...][None, :, :], (C, Ho, H))
    nb = o_ref.shape[0]
    for i in range(nb):
        zr = z_ref[0, i * C * H:(i + 1) * C * H]
        t = jnp.dot(zr, awt_ref[...], preferred_element_type=jnp.float32)
        t3 = t.astype(jnp.bfloat16).reshape(C, H, Wo)
        u = jnp.einsum('cyh,chv->cyv', ah, t3,
                       preferred_element_type=jnp.float32)
        u2 = u.reshape(C, Ho * Wo)
        o_ref[i] = u2 * sc_ref[...] + sh_ref[...]


def kernel(x_nchw, conv_w, conv_b, bn_gamma, bn_beta):
    del conv_b
    N, C_in, H, W = x_nchw.shape
    C_out = conv_w.shape[0]
    factor = 2
    Ho, Wo = H * factor, W * factor
    cnt = N * H * W
    NB = 8

    x = x_nchw.astype(jnp.float32).reshape(N, C_in, H * W)
    W2 = conv_w.reshape(C_out, C_in).astype(jnp.bfloat16)

    Z, S, Q = pl.pallas_call(
        _mix_moments_kernel,
        out_shape=(jax.ShapeDtypeStruct((N, C_out, H * W), jnp.bfloat16),
                   jax.ShapeDtypeStruct((N, C_out, 1), jnp.float32),
                   jax.ShapeDtypeStruct((N, C_out, 1), jnp.float32)),
        grid=(N // NB,),
        in_specs=[pl.BlockSpec((NB, C_in, H * W), lambda n: (n, 0, 0)),
                  pl.BlockSpec((C_out, C_in), lambda n: (0, 0))],
        out_specs=(pl.BlockSpec((NB, C_out, H * W), lambda n: (n, 0, 0)),
                   pl.BlockSpec((NB, C_out, 1), lambda n: (n, 0, 0)),
                   pl.BlockSpec((NB, C_out, 1), lambda n: (n, 0, 0))),
        compiler_params=pltpu.CompilerParams(
            dimension_semantics=("arbitrary",)),
    )(x, W2)

    mean = jnp.sum(S, axis=0)[:, 0] / cnt
    var = jnp.maximum(jnp.sum(Q, axis=0)[:, 0] / cnt - mean * mean, 0.0)
    scale = bn_gamma.astype(jnp.float32) * jax.lax.rsqrt(var + _EPS)
    shift = bn_beta.astype(jnp.float32) - scale * mean

    A_h = jnp.asarray(_bilinear_matrix(H, Ho)).astype(jnp.bfloat16)
    A_wT = jnp.asarray(_bilinear_matrix(W, Wo).T).astype(jnp.bfloat16)

    Zr = Z.reshape(N, C_out * H, W)
    flops = 2 * N * (C_out * H * W * Wo + C_out * Ho * H * Wo)
    bytes_accessed = 2 * Z.size + 4 * N * C_out * Ho * Wo

    NB2 = 2
    Zr = Zr.reshape(N // NB2, NB2 * C_out * H, W)
    out_flat = pl.pallas_call(
        _upsample_affine_kernel,
        out_shape=jax.ShapeDtypeStruct((N, C_out, Ho * Wo), jnp.float32),
        grid=(N // NB2,),
        in_specs=[
            pl.BlockSpec((1, NB2 * C_out * H, W), lambda n: (n, 0, 0)),
            pl.BlockSpec((W, Wo), lambda n: (0, 0)),
            pl.BlockSpec((Ho, H), lambda n: (0, 0)),
            pl.BlockSpec((C_out, 1), lambda n: (0, 0)),
            pl.BlockSpec((C_out, 1), lambda n: (0, 0)),
        ],
        out_specs=pl.BlockSpec((NB2, C_out, Ho * Wo), lambda n: (n, 0, 0)),
        compiler_params=pltpu.CompilerParams(
            dimension_semantics=("arbitrary",),
            vmem_limit_bytes=48 * 1024 * 1024),
        cost_estimate=pl.CostEstimate(flops=flops, transcendentals=0,
                                      bytes_accessed=bytes_accessed),
    )(Zr, A_wT, A_h, scale[:, None], shift[:, None])

    return out_flat.reshape(N, C_out, Ho, Wo)


# speedup vs baseline: 1.5765x; 1.3757x over previous
import numpy as np
import jax
import jax.numpy as jnp
from jax.experimental import pallas as pl
from jax.experimental.pallas import tpu as pltpu

_EPS = 1e-5


def _bilinear_matrix(n_in: int, n_out: int) -> np.ndarray:
    A = np.zeros((n_out, n_in), dtype=np.float32)
    if n_in == 1 or n_out == 1:
        A[:, 0] = 1.0
        return A
    src = np.arange(n_out, dtype=np.float64) * (n_in - 1) / (n_out - 1)
    lo = np.clip(np.floor(src).astype(np.int64), 0, n_in - 2)
    frac = (src - lo).astype(np.float32)
    A[np.arange(n_out), lo] += 1.0 - frac
    A[np.arange(n_out), lo + 1] += frac
    return A


def _make_body(N, C_in, C_out, H, W, Ho, Wo, NB):
    P0 = N // NB
    cnt = float(N * H * W)

    def body(x_ref, w_ref, k_ref, g_ref, b_ref, o_ref, zs_ref, s_ref, q_ref):
        i = pl.program_id(0)

        @pl.when(i < P0)
        def _phase0():
            s_tot = jnp.zeros((C_out, 1), jnp.float32)
            q_tot = jnp.zeros((C_out, 1), jnp.float32)
            for k in range(NB):
                Xb = x_ref[k].astype(jnp.bfloat16)
                Z = jnp.dot(w_ref[...], Xb,
                            preferred_element_type=jnp.float32)
                zs_ref[i * NB + k] = Z.astype(jnp.bfloat16)
                s_tot += jnp.sum(Z, axis=1, keepdims=True)
                q_tot += jnp.sum(Z * Z, axis=1, keepdims=True)

            @pl.when(i == 0)
            def _init():
                s_ref[...] = s_tot
                q_ref[...] = q_tot

            @pl.when(i > 0)
            def _acc():
                s_ref[...] += s_tot
                q_ref[...] += q_tot

        @pl.when(i >= P0)
        def _phase1():
            n = i - P0
            mean = s_ref[...] / cnt
            var = jnp.maximum(q_ref[...] / cnt - mean * mean, 0.0)
            scale = g_ref[...] * jax.lax.rsqrt(var + _EPS)
            shift = b_ref[...] - scale * mean

            z = zs_ref[n]
            u2 = jnp.dot(z, k_ref[...],
                         preferred_element_type=jnp.float32)
            o_ref[0] = u2 * scale + shift

    return body, P0


def kernel(x_nchw, conv_w, conv_b, bn_gamma, bn_beta):
    del conv_b
    N, C_in, H, W = x_nchw.shape
    C_out = conv_w.shape[0]
    factor = 2
    Ho, Wo = H * factor, W * factor
    NB = 4

    x = x_nchw.astype(jnp.float32).reshape(N, C_in, H * W)
    W2 = conv_w.reshape(C_out, C_in).astype(jnp.bfloat16)
    A_h = _bilinear_matrix(H, Ho)
    A_w = _bilinear_matrix(W, Wo)
    Kup = jnp.asarray(np.kron(A_h, A_w).T).astype(jnp.bfloat16)
    gamma = bn_gamma.astype(jnp.float32).reshape(C_out, 1)
    beta = bn_beta.astype(jnp.float32).reshape(C_out, 1)

    body, P0 = _make_body(N, C_in, C_out, H, W, Ho, Wo, NB)

    flops = 2 * N * (C_out * C_in * H * W + C_out * H * W * Ho * Wo)
    bytes_accessed = 4 * x.size + 4 * N * C_out * Ho * Wo

    out_flat = pl.pallas_call(
        body,
        out_shape=jax.ShapeDtypeStruct((N, C_out, Ho * Wo), jnp.float32),
        grid=(P0 + N,),
        in_specs=[
            pl.BlockSpec((NB, C_in, H * W),
                         lambda i: (jnp.where(i < P0, i, 0), 0, 0)),
            pl.BlockSpec((C_out, C_in), lambda i: (0, 0)),
            pl.BlockSpec((H * W, Ho * Wo), lambda i: (0, 0)),
            pl.BlockSpec((C_out, 1), lambda i: (0, 0)),
            pl.BlockSpec((C_out, 1), lambda i: (0, 0)),
        ],
        out_specs=pl.BlockSpec(
            (1, C_out, Ho * Wo),
            lambda i: (jnp.where(i < P0, 0, i - P0), 0, 0)),
        scratch_shapes=[
            pltpu.VMEM((N, C_out, H * W), jnp.bfloat16),
            pltpu.VMEM((C_out, 1), jnp.float32),
            pltpu.VMEM((C_out, 1), jnp.float32),
        ],
        compiler_params=pltpu.CompilerParams(
            dimension_semantics=("arbitrary",),
            vmem_limit_bytes=56 * 1024 * 1024),
        cost_estimate=pl.CostEstimate(flops=flops, transcendentals=0,
                                      bytes_accessed=bytes_accessed),
    )(x, W2, Kup, gamma, beta)

    return out_flat.reshape(N, C_out, Ho, Wo)


# speedup vs baseline: 1.5977x; 1.0134x over previous
import numpy as np
import jax
import jax.numpy as jnp
from jax.experimental import pallas as pl
from jax.experimental.pallas import tpu as pltpu

_EPS = 1e-5


def _bilinear_matrix(n_in: int, n_out: int) -> np.ndarray:
    A = np.zeros((n_out, n_in), dtype=np.float32)
    if n_in == 1 or n_out == 1:
        A[:, 0] = 1.0
        return A
    src = np.arange(n_out, dtype=np.float64) * (n_in - 1) / (n_out - 1)
    lo = np.clip(np.floor(src).astype(np.int64), 0, n_in - 2)
    frac = (src - lo).astype(np.float32)
    A[np.arange(n_out), lo] += 1.0 - frac
    A[np.arange(n_out), lo + 1] += frac
    return A


def _make_body(N, C_in, C_out, H, W, Ho, Wo, NB, NB2):
    P0 = N // NB
    cnt = float(N * H * W)

    def body(x_ref, w_ref, k_ref, g_ref, b_ref, o_ref, zs_ref, s_ref, q_ref):
        i = pl.program_id(0)

        @pl.when(i < P0)
        def _phase0():
            s_tot = jnp.zeros((C_out, 1), jnp.float32)
            q_tot = jnp.zeros((C_out, 1), jnp.float32)
            for k in range(NB):
                Xb = x_ref[k].astype(jnp.bfloat16)
                Z = jnp.dot(w_ref[...], Xb,
                            preferred_element_type=jnp.float32)
                zs_ref[i * NB + k] = Z.astype(jnp.bfloat16)
                s_tot += jnp.sum(Z, axis=1, keepdims=True)
                q_tot += jnp.sum(Z * Z, axis=1, keepdims=True)

            @pl.when(i == 0)
            def _init():
                s_ref[...] = s_tot
                q_ref[...] = q_tot

            @pl.when(i > 0)
            def _acc():
                s_ref[...] += s_tot
                q_ref[...] += q_tot

        @pl.when(i >= P0)
        def _phase1():
            n = i - P0
            mean = s_ref[...] / cnt
            var = jnp.maximum(q_ref[...] / cnt - mean * mean, 0.0)
            scale = g_ref[...] * jax.lax.rsqrt(var + _EPS)
            shift = b_ref[...] - scale * mean
            sc2 = jnp.concatenate([scale] * NB2, axis=0)
            sh2 = jnp.concatenate([shift] * NB2, axis=0)

            z = zs_ref[pl.ds(n * NB2, NB2)].reshape(NB2 * C_out, H * W)
            u2 = jnp.dot(z, k_ref[...],
                         preferred_element_type=jnp.float32)
            o_ref[...] = (u2 * sc2 + sh2).reshape(NB2, C_out, Ho * Wo)

    return body, P0


def kernel(x_nchw, conv_w, conv_b, bn_gamma, bn_beta):
    del conv_b
    N, C_in, H, W = x_nchw.shape
    C_out = conv_w.shape[0]
    factor = 2
    Ho, Wo = H * factor, W * factor
    NB = 4
    NB2 = 2

    x = x_nchw.astype(jnp.float32).reshape(N, C_in, H * W)
    W2 = conv_w.reshape(C_out, C_in).astype(jnp.bfloat16)
    A_h = _bilinear_matrix(H, Ho)
    A_w = _bilinear_matrix(W, Wo)
    Kup = jnp.asarray(np.kron(A_h, A_w).T).astype(jnp.bfloat16)
    gamma = bn_gamma.astype(jnp.float32).reshape(C_out, 1)
    beta = bn_beta.astype(jnp.float32).reshape(C_out, 1)

    body, P0 = _make_body(N, C_in, C_out, H, W, Ho, Wo, NB, NB2)

    flops = 2 * N * (C_out * C_in * H * W + C_out * H * W * Ho * Wo)
    bytes_accessed = 4 * x.size + 4 * N * C_out * Ho * Wo

    out_flat = pl.pallas_call(
        body,
        out_shape=jax.ShapeDtypeStruct((N, C_out, Ho * Wo), jnp.float32),
        grid=(P0 + N // NB2,),
        in_specs=[
            pl.BlockSpec((NB, C_in, H * W),
                         lambda i: (jnp.where(i < P0, i, 0), 0, 0)),
            pl.BlockSpec((C_out, C_in), lambda i: (0, 0)),
            pl.BlockSpec((H * W, Ho * Wo), lambda i: (0, 0)),
            pl.BlockSpec((C_out, 1), lambda i: (0, 0)),
            pl.BlockSpec((C_out, 1), lambda i: (0, 0)),
        ],
        out_specs=pl.BlockSpec(
            (NB2, C_out, Ho * Wo),
            lambda i: (jnp.where(i < P0, 0, i - P0), 0, 0)),
        scratch_shapes=[
            pltpu.VMEM((N, C_out, H * W), jnp.bfloat16),
            pltpu.VMEM((C_out, 1), jnp.float32),
            pltpu.VMEM((C_out, 1), jnp.float32),
        ],
        compiler_params=pltpu.CompilerParams(
            dimension_semantics=("arbitrary",),
            vmem_limit_bytes=56 * 1024 * 1024),
        cost_estimate=pl.CostEstimate(flops=flops, transcendentals=0,
                                      bytes_accessed=bytes_accessed),
    )(x, W2, Kup, gamma, beta)

    return out_flat.reshape(N, C_out, Ho, Wo)
